# Initial kernel scaffold; baseline (speedup 1.0000x reference)
import jax
import jax.numpy as jnp
from jax import lax
from jax.experimental import pallas as pl
from jax.experimental.pallas import tpu as pltpu
from jax.experimental.pallas import tpu_sc as plsc

_N = 10000
_E = 160000
_D = 256
_H = 128
_NS = 16
_NC = 2
_CH = 125
_NCH = _E // (_NS * _CH)
_RPT = _N // _NS
_BLK = 1000


def _pre_body(x_ref, w_ref, o0_ref, o1_ref):
    y = jnp.dot(x_ref[...], w_ref[...], preferred_element_type=jnp.float32,
                precision=lax.Precision.HIGHEST)
    y = jnp.maximum(y, 0.0)
    o0_ref[...] = y[:, :_H]
    o1_ref[...] = y[:, _H:]


def _pre_matmul(h_neigh, w_pre_t):
    return pl.pallas_call(
        _pre_body,
        grid=(_N // _BLK,),
        in_specs=[pl.BlockSpec((_BLK, _D), lambda i: (i, 0)),
                  pl.BlockSpec((_D, _D), lambda i: (0, 0))],
        out_specs=[pl.BlockSpec((_BLK, _H), lambda i: (i, 0)),
                   pl.BlockSpec((_BLK, _H), lambda i: (i, 0))],
        out_shape=[jax.ShapeDtypeStruct((_N, _H), jnp.float32),
                   jax.ShapeDtypeStruct((_N, _H), jnp.float32)],
    )(h_neigh, w_pre_t)


def _final_body(x_ref, s0_ref, s1_ref, c_ref, ws_t_ref, wn_t_ref, o_ref):
    inv = 1.0 / jnp.maximum(c_ref[:, 0:1], 1.0)
    wn_t = wn_t_ref[...]
    acc = jnp.dot(x_ref[...], ws_t_ref[...], preferred_element_type=jnp.float32,
                  precision=lax.Precision.HIGHEST)
    acc += jnp.dot(s0_ref[...] * inv, wn_t[:_H, :],
                   preferred_element_type=jnp.float32,
                   precision=lax.Precision.HIGHEST)
    acc += jnp.dot(s1_ref[...] * inv, wn_t[_H:, :],
                   preferred_element_type=jnp.float32,
                   precision=lax.Precision.HIGHEST)
    o_ref[...] = jnp.maximum(acc, 0.0)


def _final_update(h_self, s0, s1, cnt, ws_t, wn_t):
    return pl.pallas_call(
        _final_body,
        grid=(_N // _BLK,),
        in_specs=[pl.BlockSpec((_BLK, _D), lambda i: (i, 0)),
                  pl.BlockSpec((_BLK, _H), lambda i: (i, 0)),
                  pl.BlockSpec((_BLK, _H), lambda i: (i, 0)),
                  pl.BlockSpec((_BLK, 16), lambda i: (i, 0)),
                  pl.BlockSpec((_D, _D), lambda i: (0, 0)),
                  pl.BlockSpec((_D, _D), lambda i: (0, 0))],
        out_specs=pl.BlockSpec((_BLK, _D), lambda i: (i, 0)),
        out_shape=jax.ShapeDtypeStruct((_N, _D), jnp.float32),
    )(h_self, s0, s1, cnt, ws_t, wn_t)


def _sc_body(pre0, pre1, src_h, dst_h, zh, zc, oc,
             sum0, sum1, cnt,
             srcb, dstb, gbuf, ones, accum, cacc, sem):
    cid = lax.axis_index("c")
    sid = lax.axis_index("s")
    rows = pl.ds(sid * _RPT, _RPT)

    pltpu.sync_copy(src_h.at[pl.ds(sid * _NCH, _NCH)], srcb)
    pltpu.sync_copy(dst_h.at[pl.ds(sid * _NCH, _NCH)], dstb)
    pltpu.sync_copy(zh.at[rows], accum.at[rows])
    pltpu.sync_copy(oc, ones)

    @pl.when(cid == 0)
    def _():
        pltpu.sync_copy(zc.at[rows], cacc.at[rows])

    plsc.subcore_barrier()

    def run(table, with_counts):
        def body(j, carry):
            pltpu.async_copy(table.at[srcb.at[j]], gbuf, sem).wait()
            pltpu.sync_copy(gbuf, accum.at[dstb.at[j]], add=True)
            if with_counts:
                pltpu.sync_copy(ones, cacc.at[dstb.at[j]], add=True)
            return carry
        lax.fori_loop(0, _NCH, body, 0)

    @pl.when(cid == 0)
    def _():
        run(pre0, True)

    @pl.when(cid == 1)
    def _():
        run(pre1, False)

    plsc.subcore_barrier()

    @pl.when(cid == 0)
    def _():
        pltpu.sync_copy(accum.at[rows], sum0.at[rows])
        pltpu.sync_copy(cacc.at[rows], cnt.at[rows])

    @pl.when(cid == 1)
    def _():
        pltpu.sync_copy(accum.at[rows], sum1.at[rows])


def _sc_aggregate(pre0, pre1, src2d, dst2d, zh, zc, oc):
    f32 = jnp.float32
    kern = pl.kernel(
        _sc_body,
        out_type=[jax.ShapeDtypeStruct((_N, _H), f32),
                  jax.ShapeDtypeStruct((_N, _H), f32),
                  jax.ShapeDtypeStruct((_N, 16), f32)],
        mesh=plsc.VectorSubcoreMesh(core_axis_name="c", subcore_axis_name="s",
                                    num_cores=_NC, num_subcores=_NS),
        scratch_types=[
            pltpu.VMEM((_NCH, _CH), jnp.int32),
            pltpu.VMEM((_NCH, _CH), jnp.int32),
            pltpu.VMEM((_CH, _H), f32),
            pltpu.VMEM((_CH, 16), f32),
            pltpu.VMEM_SHARED((_N, _H), f32),
            pltpu.VMEM_SHARED((_N, 16), f32),
            pltpu.SemaphoreType.DMA,
        ],
    )
    return kern(pre0, pre1, src2d, dst2d, zh, zc, oc)


def kernel(h_neigh, h_self, edge_index, W_pre, W_self, W_neigh):
    pre0, pre1 = _pre_matmul(h_neigh, W_pre.T)
    src2d = edge_index[0].reshape(_NS * _NCH, _CH)
    dst2d = edge_index[1].reshape(_NS * _NCH, _CH)
    zh = jnp.zeros((_N, _H), jnp.float32)
    zc = jnp.zeros((_N, 16), jnp.float32)
    oc = jnp.ones((_CH, 16), jnp.float32)
    s0, s1, cnt = _sc_aggregate(pre0, pre1, src2d, dst2d, zh, zc, oc)
    return _final_update(h_self, s0, s1, cnt, W_self.T, W_neigh.T)


# speedup vs baseline: 5.7020x; 5.7020x over previous
import jax
import jax.numpy as jnp
from jax import lax
from jax.experimental import pallas as pl
from jax.experimental.pallas import tpu as pltpu
from jax.experimental.pallas import tpu_sc as plsc

_N = 10000
_E = 160000
_D = 256
_H = 128
_NS = 16
_NC = 2
_CH = 125
_GRP = 8
_NCH = _E // (_NS * _CH)
_NGRP = _NCH // _GRP
_NW = _NS * _NC
_CROWS = _E // (_NW * _CH)
_CGRP = _CROWS // _GRP
_RPA = 624
_TAIL = _N - _NS * _RPA
_BLK = 1000

_f32 = jnp.float32


def _pre_body(x_ref, w_ref, o0_ref, o1_ref):
    y = jnp.dot(x_ref[...], w_ref[...], preferred_element_type=_f32,
                precision=lax.Precision.HIGHEST)
    y = jnp.maximum(y, 0.0)
    o0_ref[...] = y[:, :_H]
    o1_ref[...] = y[:, _H:]


def _pre_matmul(h_neigh, w_pre_t):
    return pl.pallas_call(
        _pre_body,
        grid=(_N // _BLK,),
        in_specs=[pl.BlockSpec((_BLK, _D), lambda i: (i, 0)),
                  pl.BlockSpec((_D, _D), lambda i: (0, 0))],
        out_specs=[pl.BlockSpec((_BLK, _H), lambda i: (i, 0)),
                   pl.BlockSpec((_BLK, _H), lambda i: (i, 0))],
        out_shape=[jax.ShapeDtypeStruct((_N, _H), _f32),
                   jax.ShapeDtypeStruct((_N, _H), _f32)],
    )(h_neigh, w_pre_t)


def _final_body(x_ref, s0_ref, s1_ref, c0_ref, c1_ref, ws_t_ref, wn_t_ref,
                o_ref):
    count = c0_ref[:, 0:1] + c1_ref[:, 0:1]
    inv = 1.0 / jnp.maximum(count, 1.0)
    wn_t = wn_t_ref[...]
    acc = jnp.dot(x_ref[...], ws_t_ref[...], preferred_element_type=_f32,
                  precision=lax.Precision.HIGHEST)
    acc += jnp.dot(s0_ref[...] * inv, wn_t[:_H, :],
                   preferred_element_type=_f32,
                   precision=lax.Precision.HIGHEST)
    acc += jnp.dot(s1_ref[...] * inv, wn_t[_H:, :],
                   preferred_element_type=_f32,
                   precision=lax.Precision.HIGHEST)
    o_ref[...] = jnp.maximum(acc, 0.0)


def _final_update(h_self, s0, s1, cnt0, cnt1, ws_t, wn_t):
    return pl.pallas_call(
        _final_body,
        grid=(_N // _BLK,),
        in_specs=[pl.BlockSpec((_BLK, _D), lambda i: (i, 0)),
                  pl.BlockSpec((_BLK, _H), lambda i: (i, 0)),
                  pl.BlockSpec((_BLK, _H), lambda i: (i, 0)),
                  pl.BlockSpec((_BLK, 16), lambda i: (i, 0)),
                  pl.BlockSpec((_BLK, 16), lambda i: (i, 0)),
                  pl.BlockSpec((_D, _D), lambda i: (0, 0)),
                  pl.BlockSpec((_D, _D), lambda i: (0, 0))],
        out_specs=pl.BlockSpec((_BLK, _D), lambda i: (i, 0)),
        out_shape=jax.ShapeDtypeStruct((_N, _D), _f32),
    )(h_self, s0, s1, cnt0, cnt1, ws_t, wn_t)


def _sum_body(pre0, pre1, src_h, dst_h, zh,
              sum0, sum1,
              srcb, dstb, gb0, gb1, accum, sem0, sem1):
    cid = lax.axis_index("c")
    sid = lax.axis_index("s")
    rows = pl.ds(pl.multiple_of(sid * _RPA, 8), _RPA)
    tail = pl.ds(_NS * _RPA, _TAIL)
    last = sid == _NS - 1

    pltpu.sync_copy(zh.at[rows], accum.at[rows])

    @pl.when(last)
    def _():
        pltpu.sync_copy(zh.at[tail], accum.at[tail])

    plsc.subcore_barrier()

    def run(table):
        def grp(g, carry):
            row0 = pl.multiple_of(sid * _NCH + g * _GRP, 8)
            pltpu.sync_copy(src_h.at[pl.ds(row0, _GRP)], srcb)
            pltpu.sync_copy(dst_h.at[pl.ds(row0, _GRP)], dstb)
            d0 = pltpu.async_copy(table.at[srcb.at[0]], gb0, sem0)
            descs = [d0]
            for b in range(_GRP):
                gb = gb0 if b % 2 == 0 else gb1
                descs[b].wait()
                if b + 1 < _GRP:
                    nb = gb1 if b % 2 == 0 else gb0
                    ns = sem1 if b % 2 == 0 else sem0
                    descs.append(
                        pltpu.async_copy(table.at[srcb.at[b + 1]], nb, ns))
                pltpu.sync_copy(gb, accum.at[dstb.at[b]], add=True)
            return carry
        lax.fori_loop(0, _NGRP, grp, 0)

    @pl.when(cid == 0)
    def _():
        run(pre0)

    @pl.when(cid == 1)
    def _():
        run(pre1)

    plsc.subcore_barrier()

    @pl.when(cid == 0)
    def _():
        pltpu.sync_copy(accum.at[rows], sum0.at[rows])

        @pl.when(last)
        def _():
            pltpu.sync_copy(accum.at[tail], sum0.at[tail])

    @pl.when(cid == 1)
    def _():
        pltpu.sync_copy(accum.at[rows], sum1.at[rows])

        @pl.when(last)
        def _():
            pltpu.sync_copy(accum.at[tail], sum1.at[tail])


def _sc_segment_sum(pre0, pre1, src2d, dst2d, zh):
    kern = pl.kernel(
        _sum_body,
        out_type=[jax.ShapeDtypeStruct((_N, _H), _f32),
                  jax.ShapeDtypeStruct((_N, _H), _f32)],
        mesh=plsc.VectorSubcoreMesh(core_axis_name="c", subcore_axis_name="s",
                                    num_cores=_NC, num_subcores=_NS),
        scratch_types=[
            pltpu.VMEM((_GRP, _CH), jnp.int32),
            pltpu.VMEM((_GRP, _CH), jnp.int32),
            pltpu.VMEM((_CH, _H), _f32),
            pltpu.VMEM((_CH, _H), _f32),
            pltpu.VMEM_SHARED((_N, _H), _f32),
            pltpu.SemaphoreType.DMA,
            pltpu.SemaphoreType.DMA,
        ],
    )
    return kern(pre0, pre1, src2d, dst2d, zh)


def _cnt_body(dst_h, zh, oc, cnt0, cnt1, dstb, ones, cacc, sem):
    cid = lax.axis_index("c")
    sid = lax.axis_index("s")
    rows = pl.ds(pl.multiple_of(sid * _RPA, 8), _RPA)
    tail = pl.ds(_NS * _RPA, _TAIL)
    last = sid == _NS - 1

    pltpu.sync_copy(zh.at[rows], cacc.at[rows])
    pltpu.sync_copy(oc, ones)

    @pl.when(last)
    def _():
        pltpu.sync_copy(zh.at[tail], cacc.at[tail])

    plsc.subcore_barrier()

    wid = sid * _NC + cid

    def grp(g, carry):
        row0 = pl.multiple_of(wid * _CROWS + g * _GRP, 8)
        pltpu.sync_copy(dst_h.at[pl.ds(row0, _GRP)], dstb)
        for b in range(_GRP):
            pltpu.sync_copy(ones, cacc.at[dstb.at[b]], add=True)
        return carry
    lax.fori_loop(0, _CGRP, grp, 0)

    plsc.subcore_barrier()

    @pl.when(cid == 0)
    def _():
        pltpu.sync_copy(cacc.at[rows], cnt0.at[rows])

        @pl.when(last)
        def _():
            pltpu.sync_copy(cacc.at[tail], cnt0.at[tail])

    @pl.when(cid == 1)
    def _():
        pltpu.sync_copy(cacc.at[rows], cnt1.at[rows])

        @pl.when(last)
        def _():
            pltpu.sync_copy(cacc.at[tail], cnt1.at[tail])


def _sc_counts(dst2d, zh, oc):
    kern = pl.kernel(
        _cnt_body,
        out_type=[jax.ShapeDtypeStruct((_N, _H), _f32),
                  jax.ShapeDtypeStruct((_N, _H), _f32)],
        mesh=plsc.VectorSubcoreMesh(core_axis_name="c", subcore_axis_name="s",
                                    num_cores=_NC, num_subcores=_NS),
        scratch_types=[
            pltpu.VMEM((_GRP, _CH), jnp.int32),
            pltpu.VMEM((_CH, _H), _f32),
            pltpu.VMEM_SHARED((_N, _H), _f32),
            pltpu.SemaphoreType.DMA,
        ],
    )
    return kern(dst2d, zh, oc)


def kernel(h_neigh, h_self, edge_index, W_pre, W_self, W_neigh):
    pre0, pre1 = _pre_matmul(h_neigh, W_pre.T)
    src2d = edge_index[0].reshape(_NS * _NCH, _CH)
    dst2d = edge_index[1].reshape(_NS * _NCH, _CH)
    zh = jnp.zeros((_N, _H), _f32)
    oc = jnp.ones((_CH, _H), _f32)
    cnt0, cnt1 = _sc_counts(dst2d, zh, oc)
    s0, s1 = _sc_segment_sum(pre0, pre1, src2d, dst2d, zh)
    return _final_update(h_self, s0, s1, cnt0[:, :16], cnt1[:, :16],
                         W_self.T, W_neigh.T)


# speedup vs baseline: 6.0148x; 1.0549x over previous
import jax
import jax.numpy as jnp
from jax import lax
from jax.experimental import pallas as pl
from jax.experimental.pallas import tpu as pltpu
from jax.experimental.pallas import tpu_sc as plsc

_N = 10000
_E = 160000
_D = 256
_H = 128
_NS = 16
_NC = 2
_CH = 125
_NCH = _E // (_NS * _CH)
_SUP = 16
_NSUP = _NCH // _SUP
_NW = _NS * _NC
_CROWS = _E // (_NW * _CH)
_RPA = 624
_TAIL = _N - _NS * _RPA
_BLK = 1000

_f32 = jnp.float32


def _pre_body(x_ref, w_ref, o_ref):
    y = jnp.dot(x_ref[...], w_ref[...], preferred_element_type=_f32,
                precision=lax.Precision.HIGHEST)
    o_ref[...] = jnp.maximum(y, 0.0)


def _pre_matmul(h_neigh, w_pre_t):
    return pl.pallas_call(
        _pre_body,
        grid=(_N // _BLK, _NC),
        in_specs=[pl.BlockSpec((_BLK, _D), lambda i, h: (i, 0)),
                  pl.BlockSpec((_D, _H), lambda i, h: (0, h))],
        out_specs=pl.BlockSpec((_BLK, _H),
                               lambda i, h: (h * (_N // _BLK) + i, 0)),
        out_shape=jax.ShapeDtypeStruct((_NC * _N, _H), _f32),
    )(h_neigh, w_pre_t)


def _final_body(x_ref, s0_ref, s1_ref, c0_ref, c1_ref, ws_t_ref, wn_t_ref,
                o_ref):
    count = c0_ref[:, 0:1] + c1_ref[:, 0:1]
    inv = 1.0 / jnp.maximum(count, 1.0)
    wn_t = wn_t_ref[...]
    acc = jnp.dot(x_ref[...], ws_t_ref[...], preferred_element_type=_f32,
                  precision=lax.Precision.HIGHEST)
    acc += jnp.dot(s0_ref[...] * inv, wn_t[:_H, :],
                   preferred_element_type=_f32,
                   precision=lax.Precision.HIGHEST)
    acc += jnp.dot(s1_ref[...] * inv, wn_t[_H:, :],
                   preferred_element_type=_f32,
                   precision=lax.Precision.HIGHEST)
    o_ref[...] = jnp.maximum(acc, 0.0)


def _final_update(h_self, s0, s1, cnt0, cnt1, ws_t, wn_t):
    return pl.pallas_call(
        _final_body,
        grid=(_N // _BLK,),
        in_specs=[pl.BlockSpec((_BLK, _D), lambda i: (i, 0)),
                  pl.BlockSpec((_BLK, _H), lambda i: (i, 0)),
                  pl.BlockSpec((_BLK, _H), lambda i: (i, 0)),
                  pl.BlockSpec((_BLK, _H), lambda i: (i, 0)),
                  pl.BlockSpec((_BLK, _H), lambda i: (i, 0)),
                  pl.BlockSpec((_D, _D), lambda i: (0, 0)),
                  pl.BlockSpec((_D, _D), lambda i: (0, 0))],
        out_specs=pl.BlockSpec((_BLK, _D), lambda i: (i, 0)),
        out_shape=jax.ShapeDtypeStruct((_N, _D), _f32),
    )(h_self, s0, s1, cnt0, cnt1, ws_t, wn_t)


def _zero_vmem(buf):
    z16 = jnp.zeros((16,), _f32)

    def row(i, carry):
        for l in range(_H // 16):
            buf[i, pl.ds(l * 16, 16)] = z16
        return carry
    lax.fori_loop(0, _CH, row, 0)


def _spread_zero(buf, dst, sid):
    base = sid * _RPA
    for q in range(_RPA // 104):
        off = pl.multiple_of(base + q * 104, 8)
        pltpu.sync_copy(buf.at[pl.ds(0, 104)], dst.at[pl.ds(off, 104)])

    @pl.when(sid == _NS - 1)
    def _():
        pltpu.sync_copy(buf.at[pl.ds(0, _TAIL)],
                        dst.at[pl.ds(_NS * _RPA, _TAIL)])


def _copy_out(acc, out, sid):
    rows = pl.ds(pl.multiple_of(sid * _RPA, 8), _RPA)
    pltpu.sync_copy(acc.at[rows], out.at[rows])

    @pl.when(sid == _NS - 1)
    def _():
        tail = pl.ds(_NS * _RPA, _TAIL)
        pltpu.sync_copy(acc.at[tail], out.at[tail])


def _sum_body(table, srclo_h, srchi_h, dst_h,
              sum0, sum1,
              srcA, dstA, srcB, dstB, gb0, gb1, accum,
              gsem0, gsem1, ssem0, ssem1, isem):
    cid = lax.axis_index("c")
    sid = lax.axis_index("s")
    tb = sid * _NCH

    _zero_vmem(gb0)
    _spread_zero(gb0, accum, sid)

    @pl.when(cid == 0)
    def _():
        pltpu.sync_copy(srclo_h.at[pl.ds(tb, _SUP // 2)], srcA)

    @pl.when(cid == 1)
    def _():
        pltpu.sync_copy(srchi_h.at[pl.ds(tb, _SUP // 2)], srcA)

    pltpu.sync_copy(dst_h.at[pl.ds(tb, _SUP // 2)], dstA)

    plsc.subcore_barrier()

    pltpu.async_copy(table.at[srcA.at[0]], gb0, gsem0)

    def refill(dst_rows, srcb, dstb):
        row0 = pl.multiple_of(dst_rows, 8)

        @pl.when(cid == 0)
        def _():
            pltpu.async_copy(srclo_h.at[pl.ds(row0, _SUP // 2)], srcb, isem)

        @pl.when(cid == 1)
        def _():
            pltpu.async_copy(srchi_h.at[pl.ds(row0, _SUP // 2)], srcb, isem)

        pltpu.async_copy(dst_h.at[pl.ds(row0, _SUP // 2)], dstb, isem)

    def wait_refill(srcb, dstb):
        row0 = pl.multiple_of(0, 8)
        pltpu.make_async_copy(
            srclo_h.at[pl.ds(row0, _SUP // 2)], srcb, isem).wait()
        pltpu.make_async_copy(
            dst_h.at[pl.ds(row0, _SUP // 2)], dstb, isem).wait()

    def super_body(s, carry):
        for k in range(_SUP):
            r = k % (_SUP // 2)
            mysrc, mydst = (srcA, dstA) if k < 8 else (srcB, dstB)
            gb, gs = (gb0, gsem0) if k % 2 == 0 else (gb1, gsem1)
            ogb, ogs = (gb1, gsem1) if k % 2 == 0 else (gb0, gsem0)
            ss = ssem0 if k % 2 == 0 else ssem1
            osem = ssem1 if k % 2 == 0 else ssem0

            pltpu.make_async_copy(table.at[mysrc.at[r]], gb, gs).wait()

            if k == 0:
                @pl.when(s > 0)
                def _():
                    pltpu.make_async_copy(
                        gb1, accum.at[dstB.at[7]], ssem1).wait()
            else:
                pdst = dstA if (k - 1) < 8 else dstB
                pltpu.make_async_copy(
                    ogb, accum.at[pdst.at[(k - 1) % 8]], osem).wait()

            if k == 0:
                refill(tb + s * _SUP + 8, srcB, dstB)
            if k == 8:
                @pl.when(s < _NSUP - 1)
                def _():
                    refill(tb + (s + 1) * _SUP, srcA, dstA)

            if k < _SUP - 1:
                nk = k + 1
                nsrc = srcB if nk >= 8 else srcA
                if nk == 8:
                    wait_refill(srcB, dstB)
                pltpu.async_copy(table.at[nsrc.at[nk % 8]], ogb, ogs)
            else:
                @pl.when(s < _NSUP - 1)
                def _():
                    wait_refill(srcA, dstA)
                    pltpu.async_copy(table.at[srcA.at[0]], ogb, ogs)

            pltpu.async_copy(gb, accum.at[mydst.at[r]], ss, add=True)
        return carry

    lax.fori_loop(0, _NSUP, super_body, 0)

    pltpu.make_async_copy(gb1, accum.at[dstB.at[7]], ssem1).wait()

    plsc.subcore_barrier()

    @pl.when(cid == 0)
    def _():
        _copy_out(accum, sum0, sid)

    @pl.when(cid == 1)
    def _():
        _copy_out(accum, sum1, sid)


def _sc_segment_sum(pre_cat, srclo2d, srchi2d, dst2d):
    kern = pl.kernel(
        _sum_body,
        out_type=[jax.ShapeDtypeStruct((_N, _H), _f32),
                  jax.ShapeDtypeStruct((_N, _H), _f32)],
        mesh=plsc.VectorSubcoreMesh(core_axis_name="c", subcore_axis_name="s",
                                    num_cores=_NC, num_subcores=_NS),
        scratch_types=[
            pltpu.VMEM((_SUP // 2, _CH), jnp.int32),
            pltpu.VMEM((_SUP // 2, _CH), jnp.int32),
            pltpu.VMEM((_SUP // 2, _CH), jnp.int32),
            pltpu.VMEM((_SUP // 2, _CH), jnp.int32),
            pltpu.VMEM((_CH, _H), _f32),
            pltpu.VMEM((_CH, _H), _f32),
            pltpu.VMEM_SHARED((_N, _H), _f32),
            pltpu.SemaphoreType.DMA,
            pltpu.SemaphoreType.DMA,
            pltpu.SemaphoreType.DMA,
            pltpu.SemaphoreType.DMA,
            pltpu.SemaphoreType.DMA,
        ],
    )
    return kern(pre_cat, srclo2d, srchi2d, dst2d)


def _cnt_body(dst_h, cnt0, cnt1, dstb, ones, zbuf, cacc, ssem):
    cid = lax.axis_index("c")
    sid = lax.axis_index("s")

    one16 = jnp.ones((16,), _f32)

    def row(i, carry):
        for l in range(_H // 16):
            ones[i, pl.ds(l * 16, 16)] = one16
        return carry
    lax.fori_loop(0, _CH, row, 0)
    _zero_vmem(zbuf)
    _spread_zero(zbuf, cacc, sid)

    wid = sid * _NC + cid
    pltpu.sync_copy(dst_h.at[pl.ds(pl.multiple_of(wid * _CROWS, 8), _CROWS)],
                    dstb)

    plsc.subcore_barrier()

    def grp(g, carry):
        descs = []
        for b in range(8):
            jj = g * 8 + b
            descs.append(
                pltpu.async_copy(ones, cacc.at[dstb.at[jj]], ssem, add=True))
        for d in descs:
            d.wait()
        return carry
    lax.fori_loop(0, _CROWS // 8, grp, 0)

    plsc.subcore_barrier()

    @pl.when(cid == 0)
    def _():
        _copy_out(cacc, cnt0, sid)

    @pl.when(cid == 1)
    def _():
        _copy_out(cacc, cnt1, sid)


def _sc_counts(dst2d):
    kern = pl.kernel(
        _cnt_body,
        out_type=[jax.ShapeDtypeStruct((_N, _H), _f32),
                  jax.ShapeDtypeStruct((_N, _H), _f32)],
        mesh=plsc.VectorSubcoreMesh(core_axis_name="c", subcore_axis_name="s",
                                    num_cores=_NC, num_subcores=_NS),
        scratch_types=[
            pltpu.VMEM((_CROWS, _CH), jnp.int32),
            pltpu.VMEM((_CH, _H), _f32),
            pltpu.VMEM((_CH, _H), _f32),
            pltpu.VMEM_SHARED((_N, _H), _f32),
            pltpu.SemaphoreType.DMA,
        ],
    )
    return kern(dst2d)


def kernel(h_neigh, h_self, edge_index, W_pre, W_self, W_neigh):
    pre_cat = _pre_matmul(h_neigh, W_pre.T)
    src = edge_index[0]
    srclo2d = src.reshape(_NS * _NCH, _CH)
    srchi2d = (src + _N).reshape(_NS * _NCH, _CH)
    dst2d = edge_index[1].reshape(_NS * _NCH, _CH)
    cnt0, cnt1 = _sc_counts(dst2d)
    s0, s1 = _sc_segment_sum(pre_cat, srclo2d, srchi2d, dst2d)
    return _final_update(h_self, s0, s1, cnt0, cnt1, W_self.T, W_neigh.T)
